# baseline (speedup 1.0000x reference)
import functools

import jax
import jax.numpy as jnp
from jax import lax
from jax.experimental import pallas as pl
from jax.experimental.pallas import tpu as pltpu
from jax.experimental.pallas import tpu_sc as plsc

N = 10000
NP = 10240
E = 160000
EP = 163840
D = 256
HD = 128
CH = 128
CH1 = 32
G = 32
RPW = NP // 16
NCH1 = EP // 16 // CH1
NG1 = NCH1 // G
NCH2 = EP // 32 // CH
RB = 1000


def _tc1_body(x_ref, w1lt_ref, w1rt_ref, b1l_ref, y1a_ref, y1b_ref, r1_ref):
    xb = x_ref[...]
    y1 = jnp.dot(xb, w1lt_ref[...], preferred_element_type=jnp.float32)
    y1a_ref[...] = y1[:, :HD]
    y1b_ref[...] = y1[:, HD:]
    r1_ref[...] = (
        jnp.dot(xb, w1rt_ref[...], preferred_element_type=jnp.float32)
        + b1l_ref[...]
    )


def _tc1(x, w1lt, w1rt, b1l2d):
    return pl.pallas_call(
        _tc1_body,
        grid=(N // RB,),
        in_specs=[
            pl.BlockSpec((RB, D), lambda i: (i, 0)),
            pl.BlockSpec((D, D), lambda i: (0, 0)),
            pl.BlockSpec((D, D), lambda i: (0, 0)),
            pl.BlockSpec((1, D), lambda i: (0, 0)),
        ],
        out_specs=[
            pl.BlockSpec((RB, HD), lambda i: (i, 0)),
            pl.BlockSpec((RB, HD), lambda i: (i, 0)),
            pl.BlockSpec((RB, D), lambda i: (i, 0)),
        ],
        out_shape=[
            jax.ShapeDtypeStruct((N, HD), jnp.float32),
            jax.ShapeDtypeStruct((N, HD), jnp.float32),
            jax.ShapeDtypeStruct((N, D), jnp.float32),
        ],
    )(x, w1lt, w1rt, b1l2d)


def _sc1_body(y1a, y1b, src2d, dst2d, z128, z16, ones16,
              agga_o, aggb_o, dega_o, degb_o,
              agg_sh, deg_sh, sblk, dblk, buf0, buf1, buf2, buf3, buf4,
              ones_v, isem, gsem0, gsem1, gsem2, gsem3, gsem4,
              ssem0, ssem1, ssem2, ssem3, ssem4, dsem):
    c = lax.axis_index("c")
    s = lax.axis_index("s")
    rbase = s * RPW
    pltpu.sync_copy(z128, buf0)
    pltpu.sync_copy(z16, ones_v)
    for j in range(RPW // CH1):
        pltpu.sync_copy(buf0, agg_sh.at[pl.ds(rbase + j * CH1, CH1)])
        pltpu.sync_copy(ones_v, deg_sh.at[pl.ds(rbase + j * CH1, CH1)])

    pltpu.sync_copy(ones16, ones_v)
    cb = s * NCH1
    plsc.subcore_barrier()

    bufs = (buf0, buf1, buf2, buf3, buf4)
    gsems = (gsem0, gsem1, gsem2, gsem3, gsem4)
    ssems = (ssem0, ssem1, ssem2, ssem3, ssem4)
    R = 5
    K = 2

    def run(table):
        def body(gi, carry):
            gb = cb + gi * G
            di1 = pltpu.async_copy(src2d.at[pl.ds(gb, G)], sblk, isem)
            di2 = pltpu.async_copy(dst2d.at[pl.ds(gb, G)], dblk, isem)
            di1.wait()
            di2.wait()

            @pl.when((gi % 2) == c)
            def _():
                for j in range(G):
                    pltpu.async_copy(ones_v, deg_sh.at[dblk.at[j]], dsem,
                                     add=True)

            gds = {}
            sds = {}
            for m in range(K):
                gds[m] = pltpu.async_copy(table.at[sblk.at[m]], bufs[m % R],
                                          gsems[m % R])
            for j in range(G):
                t = j + K
                if t < G:
                    if t - R >= 0:
                        sds[t - R].wait()
                    gds[t] = pltpu.async_copy(table.at[sblk.at[t]],
                                              bufs[t % R], gsems[t % R])
                gds[j].wait()
                sds[j] = pltpu.async_copy(bufs[j % R],
                                          agg_sh.at[dblk.at[j]],
                                          ssems[j % R], add=True)
            for m in range(G - R, G):
                sds[m].wait()

            @pl.when((gi % 2) == c)
            def _():
                for j in range(G):
                    pltpu.make_async_copy(z16, ones_v, dsem).wait()

            return carry

        lax.fori_loop(0, NG1, body, 0)

    @pl.when(c == 0)
    def _():
        run(y1a)

    @pl.when(c == 1)
    def _():
        run(y1b)

    plsc.subcore_barrier()
    for j in range(RPW // CH1):
        rs = rbase + j * CH1
        pltpu.sync_copy(agg_sh.at[pl.ds(rs, CH1)], buf0)

        @pl.when(c == 0)
        def _():
            pltpu.sync_copy(deg_sh.at[pl.ds(rs, CH1)], ones_v)
            pltpu.sync_copy(ones_v, dega_o.at[pl.ds(rs, CH1)])
            pltpu.sync_copy(buf0, agga_o.at[pl.ds(rs, CH1)])

        @pl.when(c == 1)
        def _():
            pltpu.sync_copy(deg_sh.at[pl.ds(rs, CH1)], ones_v)
            pltpu.sync_copy(ones_v, degb_o.at[pl.ds(rs, CH1)])
            pltpu.sync_copy(buf0, aggb_o.at[pl.ds(rs, CH1)])


def _sc1(y1a, y1b, src2d, dst2d, z128, z16, ones16):
    mesh = plsc.VectorSubcoreMesh(core_axis_name="c", subcore_axis_name="s", num_cores=2, num_subcores=16)
    return pl.kernel(
        _sc1_body,
        out_type=[
            jax.ShapeDtypeStruct((NP, HD), jnp.float32),
            jax.ShapeDtypeStruct((NP, HD), jnp.float32),
            jax.ShapeDtypeStruct((NP, 16), jnp.float32),
            jax.ShapeDtypeStruct((NP, 16), jnp.float32),
        ],
        mesh=mesh,
        scratch_types=[
            pltpu.VMEM_SHARED((NP, HD), jnp.float32),
            pltpu.VMEM_SHARED((NP, 16), jnp.float32),
            pltpu.VMEM((G, CH1), jnp.int32),
            pltpu.VMEM((G, CH1), jnp.int32),
            pltpu.VMEM((CH1, HD), jnp.float32),
            pltpu.VMEM((CH1, HD), jnp.float32),
            pltpu.VMEM((CH1, HD), jnp.float32),
            pltpu.VMEM((CH1, HD), jnp.float32),
            pltpu.VMEM((CH1, HD), jnp.float32),
            pltpu.VMEM((CH1, 16), jnp.float32),
        ] + [pltpu.SemaphoreType.DMA] * 12,
        compiler_params=pltpu.CompilerParams(use_tc_tiling_on_sc=False),
    )(y1a, y1b, src2d, dst2d, z128, z16, ones16)


def _tc2_body(agga_ref, aggb_ref, dega_ref, degb_ref, r1_ref, w2cat_ref,
              bcat_ref, y2_ref, base_ref):
    rdeg = 1.0 / jnp.maximum(dega_ref[:, 0:1] + degb_ref[:, 0:1], 1.0)
    h0 = jnp.maximum(agga_ref[...] * rdeg + r1_ref[:, :HD], 0.0)
    h1 = jnp.maximum(aggb_ref[...] * rdeg + r1_ref[:, HD:], 0.0)
    res = (
        jnp.dot(h0, w2cat_ref[:HD, :], preferred_element_type=jnp.float32)
        + jnp.dot(h1, w2cat_ref[HD:, :], preferred_element_type=jnp.float32)
        + bcat_ref[...]
    )
    y2_ref[...] = res[:, :16]
    base_ref[...] = res[:, 16:32]


def _tc2(agga, aggb, dega, degb, r1, w2cat, bcat):
    return pl.pallas_call(
        _tc2_body,
        grid=(N // RB,),
        in_specs=[
            pl.BlockSpec((RB, HD), lambda i: (i, 0)),
            pl.BlockSpec((RB, HD), lambda i: (i, 0)),
            pl.BlockSpec((RB, 16), lambda i: (i, 0)),
            pl.BlockSpec((RB, 16), lambda i: (i, 0)),
            pl.BlockSpec((RB, D), lambda i: (i, 0)),
            pl.BlockSpec((D, 128), lambda i: (0, 0)),
            pl.BlockSpec((1, 128), lambda i: (0, 0)),
        ],
        out_specs=[
            pl.BlockSpec((RB, 16), lambda i: (i, 0)),
            pl.BlockSpec((RB, 16), lambda i: (i, 0)),
        ],
        out_shape=[
            jax.ShapeDtypeStruct((N, 16), jnp.float32),
            jax.ShapeDtypeStruct((N, 16), jnp.float32),
        ],
    )(agga, aggb, dega, degb, r1, w2cat, bcat)


def _sc2_body(y2, src2d, dst2d, z16,
              agg2a_o, agg2b_o,
              agg_sh, src_v, dst_v, rows_v, sem):
    c = lax.axis_index("c")
    s = lax.axis_index("s")
    rbase = s * RPW
    pltpu.sync_copy(z16, rows_v)
    for j in range(RPW // CH):
        pltpu.sync_copy(rows_v, agg_sh.at[pl.ds(rbase + j * CH, CH)])
    w = c * 16 + s
    cb = w * NCH2
    pltpu.sync_copy(src2d.at[pl.ds(cb, NCH2)], src_v)
    pltpu.sync_copy(dst2d.at[pl.ds(cb, NCH2)], dst_v)
    plsc.subcore_barrier()

    def body(i, carry):
        pltpu.async_copy(y2.at[src_v.at[i]], rows_v, sem).wait()
        pltpu.sync_copy(rows_v, agg_sh.at[dst_v.at[i]], add=True)
        return carry

    lax.fori_loop(0, NCH2, body, 0)
    plsc.subcore_barrier()

    for j in range(RPW // CH):
        rs = rbase + j * CH
        pltpu.sync_copy(agg_sh.at[pl.ds(rs, CH)], rows_v)

        @pl.when(c == 0)
        def _():
            pltpu.sync_copy(rows_v, agg2a_o.at[pl.ds(rs, CH)])

        @pl.when(c == 1)
        def _():
            pltpu.sync_copy(rows_v, agg2b_o.at[pl.ds(rs, CH)])


def _sc2(y2, src2d, dst2d, z16):
    mesh = plsc.VectorSubcoreMesh(core_axis_name="c", subcore_axis_name="s", num_cores=2, num_subcores=16)
    return pl.kernel(
        _sc2_body,
        out_type=[
            jax.ShapeDtypeStruct((NP, 16), jnp.float32),
            jax.ShapeDtypeStruct((NP, 16), jnp.float32),
        ],
        mesh=mesh,
        scratch_types=[
            pltpu.VMEM_SHARED((NP, 16), jnp.float32),
            pltpu.VMEM((NCH2, CH), jnp.int32),
            pltpu.VMEM((NCH2, CH), jnp.int32),
            pltpu.VMEM((CH, 16), jnp.float32),
            pltpu.SemaphoreType.DMA,
        ],
        compiler_params=pltpu.CompilerParams(use_tc_tiling_on_sc=False),
    )(y2, src2d, dst2d, z16)


def _tc3_body(a2a_ref, a2b_ref, dega_ref, degb_ref, base_ref, out_ref):
    rdeg = 1.0 / jnp.maximum(dega_ref[:, 0:1] + degb_ref[:, 0:1], 1.0)
    o = (a2a_ref[...] + a2b_ref[...]) * rdeg + base_ref[...]
    sg = jax.nn.sigmoid(o)
    fsi = jnp.maximum(o[:, 0:1], 0.0) + sg[:, 1:2]
    out_ref[...] = jnp.concatenate([fsi, sg[:, 1:2], sg[:, 2:3]], axis=1)


def _tc3(a2a, a2b, dega, degb, base):
    return pl.pallas_call(
        _tc3_body,
        grid=(N // RB,),
        in_specs=[
            pl.BlockSpec((RB, 16), lambda i: (i, 0)),
            pl.BlockSpec((RB, 16), lambda i: (i, 0)),
            pl.BlockSpec((RB, 16), lambda i: (i, 0)),
            pl.BlockSpec((RB, 16), lambda i: (i, 0)),
            pl.BlockSpec((RB, 16), lambda i: (i, 0)),
        ],
        out_specs=pl.BlockSpec((RB, 3), lambda i: (i, 0)),
        out_shape=jax.ShapeDtypeStruct((N, 3), jnp.float32),
    )(a2a, a2b, dega, degb, base)


def kernel(x, edge_index, W1l, b1l, W1r, W2l, b2l, W2r):
    src = edge_index[0].astype(jnp.int32)
    dst = edge_index[1].astype(jnp.int32)
    pad = EP - E
    srcp = jnp.concatenate([src, jnp.zeros((pad,), jnp.int32)])
    dstp = jnp.concatenate([dst, jnp.full((pad,), N, jnp.int32)])
    src2da = srcp.reshape(EP // CH1, CH1)
    dst2da = dstp.reshape(EP // CH1, CH1)
    src2db = srcp.reshape(EP // CH, CH)
    dst2db = dstp.reshape(EP // CH, CH)

    w1lt = W1l.T
    w1rt = W1r.T
    b1l2d = b1l[None, :]
    w2cat = jnp.zeros((D, 128), jnp.float32)
    w2cat = w2cat.at[:, 0:3].set(W2l.T).at[:, 16:19].set(W2r.T)
    bcat = jnp.zeros((1, 128), jnp.float32).at[0, 16:19].set(b2l)

    z128 = jnp.zeros((CH1, HD), jnp.float32)
    z16 = jnp.zeros((CH1, 16), jnp.float32)
    z16b = jnp.zeros((CH, 16), jnp.float32)
    ones16 = jnp.ones((CH1, 16), jnp.float32)

    y1a, y1b, r1 = _tc1(x, w1lt, w1rt, b1l2d)
    agga, aggb, dega, degb = _sc1(y1a, y1b, src2da, dst2da, z128, z16, ones16)
    y2, base = _tc2(agga, aggb, dega, degb, r1, w2cat, bcat)
    agg2a, agg2b = _sc2(y2, src2db, dst2db, z16b)
    return _tc3(agg2a, agg2b, dega, degb, base)


# speedup vs baseline: 1.0456x; 1.0456x over previous
import functools

import jax
import jax.numpy as jnp
from jax import lax
from jax.experimental import pallas as pl
from jax.experimental.pallas import tpu as pltpu
from jax.experimental.pallas import tpu_sc as plsc

N = 10000
NP = 10240
E = 160000
EP = 163840
D = 256
HD = 128
CH = 128
CH1 = 32
G = 32
RPW = NP // 16
NCH1 = EP // 16 // CH1
NG1 = NCH1 // G
NCH2 = EP // 32 // CH
RB = 1000


def _tc1_body(x_ref, w1lt_ref, w1rt_ref, b1l_ref, y1a_ref, y1b_ref, r1_ref):
    xb = x_ref[...]
    y1 = jnp.dot(xb, w1lt_ref[...], preferred_element_type=jnp.float32)
    y1a_ref[...] = y1[:, :HD]
    y1b_ref[...] = y1[:, HD:]
    r1_ref[...] = (
        jnp.dot(xb, w1rt_ref[...], preferred_element_type=jnp.float32)
        + b1l_ref[...]
    )


def _tc1(x, w1lt, w1rt, b1l2d):
    return pl.pallas_call(
        _tc1_body,
        grid=(N // RB,),
        in_specs=[
            pl.BlockSpec((RB, D), lambda i: (i, 0)),
            pl.BlockSpec((D, D), lambda i: (0, 0)),
            pl.BlockSpec((D, D), lambda i: (0, 0)),
            pl.BlockSpec((1, D), lambda i: (0, 0)),
        ],
        out_specs=[
            pl.BlockSpec((RB, HD), lambda i: (i, 0)),
            pl.BlockSpec((RB, HD), lambda i: (i, 0)),
            pl.BlockSpec((RB, D), lambda i: (i, 0)),
        ],
        out_shape=[
            jax.ShapeDtypeStruct((N, HD), jnp.float32),
            jax.ShapeDtypeStruct((N, HD), jnp.float32),
            jax.ShapeDtypeStruct((N, D), jnp.float32),
        ],
    )(x, w1lt, w1rt, b1l2d)


def _sc1_body(y1a, y1b, src2d, dst2d, z128, z16, ones16,
              agga_o, aggb_o, dega_o, degb_o,
              agg_sh, deg_sh, sblk, dblk, buf0, buf1, buf2, buf3, ones_v,
              isem, gsem0, gsem1, gsem2, gsem3, ssem0, ssem1, ssem2, ssem3,
              dsem):
    c = lax.axis_index("c")
    s = lax.axis_index("s")
    rbase = s * RPW
    pltpu.sync_copy(z128, buf0)
    pltpu.sync_copy(z16, ones_v)
    for j in range(RPW // CH1):
        pltpu.sync_copy(buf0, agg_sh.at[pl.ds(rbase + j * CH1, CH1)])
        pltpu.sync_copy(ones_v, deg_sh.at[pl.ds(rbase + j * CH1, CH1)])

    pltpu.sync_copy(ones16, ones_v)
    cb = s * NCH1
    plsc.subcore_barrier()

    bufs = (buf0, buf1, buf2, buf3)
    gsems = (gsem0, gsem1, gsem2, gsem3)
    ssems = (ssem0, ssem1, ssem2, ssem3)

    def run(table):
        def body(gi, carry):
            gb = cb + gi * G
            di1 = pltpu.async_copy(src2d.at[pl.ds(gb, G)], sblk, isem)
            di2 = pltpu.async_copy(dst2d.at[pl.ds(gb, G)], dblk, isem)
            di1.wait()
            di2.wait()

            @pl.when((gi % 2) == c)
            def _():
                for j in range(G):
                    pltpu.async_copy(ones_v, deg_sh.at[dblk.at[j]], dsem,
                                     add=True)

            gds = {}
            sds = {}
            gds[0] = pltpu.async_copy(table.at[sblk.at[0]], bufs[0], gsems[0])
            gds[1] = pltpu.async_copy(table.at[sblk.at[1]], bufs[1], gsems[1])
            for j in range(G):
                if j + 2 < G:
                    if j - 2 >= 0:
                        sds[j - 2].wait()
                    r = (j + 2) % 4
                    gds[j + 2] = pltpu.async_copy(table.at[sblk.at[j + 2]],
                                                  bufs[r], gsems[r])
                gds[j].wait()
                sds[j] = pltpu.async_copy(bufs[j % 4],
                                          agg_sh.at[dblk.at[j]],
                                          ssems[j % 4], add=True)
            sds[G - 4].wait()
            sds[G - 3].wait()
            sds[G - 2].wait()
            sds[G - 1].wait()

            @pl.when((gi % 2) == c)
            def _():
                for j in range(G):
                    pltpu.make_async_copy(z16, ones_v, dsem).wait()

            return carry

        lax.fori_loop(0, NG1, body, 0)

    @pl.when(c == 0)
    def _():
        run(y1a)

    @pl.when(c == 1)
    def _():
        run(y1b)

    plsc.subcore_barrier()
    for j in range(RPW // CH1):
        rs = rbase + j * CH1
        pltpu.sync_copy(agg_sh.at[pl.ds(rs, CH1)], buf0)

        @pl.when(c == 0)
        def _():
            pltpu.sync_copy(deg_sh.at[pl.ds(rs, CH1)], ones_v)
            pltpu.sync_copy(ones_v, dega_o.at[pl.ds(rs, CH1)])
            pltpu.sync_copy(buf0, agga_o.at[pl.ds(rs, CH1)])

        @pl.when(c == 1)
        def _():
            pltpu.sync_copy(deg_sh.at[pl.ds(rs, CH1)], ones_v)
            pltpu.sync_copy(ones_v, degb_o.at[pl.ds(rs, CH1)])
            pltpu.sync_copy(buf0, aggb_o.at[pl.ds(rs, CH1)])


def _sc1(y1a, y1b, src2d, dst2d, z128, z16, ones16):
    mesh = plsc.VectorSubcoreMesh(core_axis_name="c", subcore_axis_name="s", num_cores=2, num_subcores=16)
    return pl.kernel(
        _sc1_body,
        out_type=[
            jax.ShapeDtypeStruct((NP, HD), jnp.float32),
            jax.ShapeDtypeStruct((NP, HD), jnp.float32),
            jax.ShapeDtypeStruct((NP, 16), jnp.float32),
            jax.ShapeDtypeStruct((NP, 16), jnp.float32),
        ],
        mesh=mesh,
        scratch_types=[
            pltpu.VMEM_SHARED((NP, HD), jnp.float32),
            pltpu.VMEM_SHARED((NP, 16), jnp.float32),
            pltpu.VMEM((G, CH1), jnp.int32),
            pltpu.VMEM((G, CH1), jnp.int32),
            pltpu.VMEM((CH1, HD), jnp.float32),
            pltpu.VMEM((CH1, HD), jnp.float32),
            pltpu.VMEM((CH1, HD), jnp.float32),
            pltpu.VMEM((CH1, HD), jnp.float32),
            pltpu.VMEM((CH1, 16), jnp.float32),
            pltpu.SemaphoreType.DMA,
            pltpu.SemaphoreType.DMA,
            pltpu.SemaphoreType.DMA,
            pltpu.SemaphoreType.DMA,
            pltpu.SemaphoreType.DMA,
            pltpu.SemaphoreType.DMA,
            pltpu.SemaphoreType.DMA,
            pltpu.SemaphoreType.DMA,
            pltpu.SemaphoreType.DMA,
            pltpu.SemaphoreType.DMA,
        ],
        compiler_params=pltpu.CompilerParams(use_tc_tiling_on_sc=False),
    )(y1a, y1b, src2d, dst2d, z128, z16, ones16)


def _tc2_body(agga_ref, aggb_ref, dega_ref, degb_ref, r1_ref, w2cat_ref,
              bcat_ref, y2_ref, base_ref):
    rdeg = 1.0 / jnp.maximum(dega_ref[:, 0:1] + degb_ref[:, 0:1], 1.0)
    h0 = jnp.maximum(agga_ref[...] * rdeg + r1_ref[:, :HD], 0.0)
    h1 = jnp.maximum(aggb_ref[...] * rdeg + r1_ref[:, HD:], 0.0)
    res = (
        jnp.dot(h0, w2cat_ref[:HD, :], preferred_element_type=jnp.float32)
        + jnp.dot(h1, w2cat_ref[HD:, :], preferred_element_type=jnp.float32)
        + bcat_ref[...]
    )
    y2_ref[...] = res[:, :16]
    base_ref[...] = res[:, 16:32]


def _tc2(agga, aggb, dega, degb, r1, w2cat, bcat):
    return pl.pallas_call(
        _tc2_body,
        grid=(N // RB,),
        in_specs=[
            pl.BlockSpec((RB, HD), lambda i: (i, 0)),
            pl.BlockSpec((RB, HD), lambda i: (i, 0)),
            pl.BlockSpec((RB, 16), lambda i: (i, 0)),
            pl.BlockSpec((RB, 16), lambda i: (i, 0)),
            pl.BlockSpec((RB, D), lambda i: (i, 0)),
            pl.BlockSpec((D, 128), lambda i: (0, 0)),
            pl.BlockSpec((1, 128), lambda i: (0, 0)),
        ],
        out_specs=[
            pl.BlockSpec((RB, 16), lambda i: (i, 0)),
            pl.BlockSpec((RB, 16), lambda i: (i, 0)),
        ],
        out_shape=[
            jax.ShapeDtypeStruct((N, 16), jnp.float32),
            jax.ShapeDtypeStruct((N, 16), jnp.float32),
        ],
    )(agga, aggb, dega, degb, r1, w2cat, bcat)


def _sc2_body(y2, src2d, dst2d, z16,
              agg2a_o, agg2b_o,
              agg_sh, sblk, dblk, b0, b1, b2, b3,
              isem, qsem0, qsem1, qsem2, qsem3, tsem0, tsem1, tsem2, tsem3):
    c = lax.axis_index("c")
    s = lax.axis_index("s")
    rbase = s * RPW
    pltpu.sync_copy(z16, b0)
    for j in range(RPW // CH):
        pltpu.sync_copy(b0, agg_sh.at[pl.ds(rbase + j * CH, CH)])
    w = c * 16 + s
    cb = w * NCH2
    di1 = pltpu.async_copy(src2d.at[pl.ds(cb, NCH2)], sblk, isem)
    di2 = pltpu.async_copy(dst2d.at[pl.ds(cb, NCH2)], dblk, isem)
    di1.wait()
    di2.wait()
    plsc.subcore_barrier()

    bufs = (b0, b1, b2, b3)
    qsems = (qsem0, qsem1, qsem2, qsem3)
    tsems = (tsem0, tsem1, tsem2, tsem3)
    R = 4
    K = 2
    gds = {}
    sds = {}
    for m in range(K):
        gds[m] = pltpu.async_copy(y2.at[sblk.at[m]], bufs[m % R], qsems[m % R])
    for j in range(NCH2):
        t = j + K
        if t < NCH2:
            if t - R >= 0:
                sds[t - R].wait()
            gds[t] = pltpu.async_copy(y2.at[sblk.at[t]], bufs[t % R],
                                      qsems[t % R])
        gds[j].wait()
        sds[j] = pltpu.async_copy(bufs[j % R], agg_sh.at[dblk.at[j]],
                                  tsems[j % R], add=True)
    for m in range(max(0, NCH2 - R), NCH2):
        sds[m].wait()
    plsc.subcore_barrier()

    for j in range(RPW // CH):
        rs = rbase + j * CH
        pltpu.sync_copy(agg_sh.at[pl.ds(rs, CH)], b0)

        @pl.when(c == 0)
        def _():
            pltpu.sync_copy(b0, agg2a_o.at[pl.ds(rs, CH)])

        @pl.when(c == 1)
        def _():
            pltpu.sync_copy(b0, agg2b_o.at[pl.ds(rs, CH)])


def _sc2(y2, src2d, dst2d, z16):
    mesh = plsc.VectorSubcoreMesh(core_axis_name="c", subcore_axis_name="s", num_cores=2, num_subcores=16)
    return pl.kernel(
        _sc2_body,
        out_type=[
            jax.ShapeDtypeStruct((NP, 16), jnp.float32),
            jax.ShapeDtypeStruct((NP, 16), jnp.float32),
        ],
        mesh=mesh,
        scratch_types=[
            pltpu.VMEM_SHARED((NP, 16), jnp.float32),
            pltpu.VMEM((NCH2, CH), jnp.int32),
            pltpu.VMEM((NCH2, CH), jnp.int32),
            pltpu.VMEM((CH, 16), jnp.float32),
            pltpu.VMEM((CH, 16), jnp.float32),
            pltpu.VMEM((CH, 16), jnp.float32),
            pltpu.VMEM((CH, 16), jnp.float32),
        ] + [pltpu.SemaphoreType.DMA] * 9,
        compiler_params=pltpu.CompilerParams(use_tc_tiling_on_sc=False),
    )(y2, src2d, dst2d, z16)


def _tc3_body(a2a_ref, a2b_ref, dega_ref, degb_ref, base_ref, out_ref):
    rdeg = 1.0 / jnp.maximum(dega_ref[:, 0:1] + degb_ref[:, 0:1], 1.0)
    o = (a2a_ref[...] + a2b_ref[...]) * rdeg + base_ref[...]
    sg = jax.nn.sigmoid(o)
    fsi = jnp.maximum(o[:, 0:1], 0.0) + sg[:, 1:2]
    out_ref[...] = jnp.concatenate([fsi, sg[:, 1:2], sg[:, 2:3]], axis=1)


def _tc3(a2a, a2b, dega, degb, base):
    return pl.pallas_call(
        _tc3_body,
        grid=(N // RB,),
        in_specs=[
            pl.BlockSpec((RB, 16), lambda i: (i, 0)),
            pl.BlockSpec((RB, 16), lambda i: (i, 0)),
            pl.BlockSpec((RB, 16), lambda i: (i, 0)),
            pl.BlockSpec((RB, 16), lambda i: (i, 0)),
            pl.BlockSpec((RB, 16), lambda i: (i, 0)),
        ],
        out_specs=pl.BlockSpec((RB, 3), lambda i: (i, 0)),
        out_shape=jax.ShapeDtypeStruct((N, 3), jnp.float32),
    )(a2a, a2b, dega, degb, base)


def kernel(x, edge_index, W1l, b1l, W1r, W2l, b2l, W2r):
    src = edge_index[0].astype(jnp.int32)
    dst = edge_index[1].astype(jnp.int32)
    pad = EP - E
    srcp = jnp.concatenate([src, jnp.zeros((pad,), jnp.int32)])
    dstp = jnp.concatenate([dst, jnp.full((pad,), N, jnp.int32)])
    src2da = srcp.reshape(EP // CH1, CH1)
    dst2da = dstp.reshape(EP // CH1, CH1)
    src2db = srcp.reshape(EP // CH, CH)
    dst2db = dstp.reshape(EP // CH, CH)

    w1lt = W1l.T
    w1rt = W1r.T
    b1l2d = b1l[None, :]
    w2cat = jnp.zeros((D, 128), jnp.float32)
    w2cat = w2cat.at[:, 0:3].set(W2l.T).at[:, 16:19].set(W2r.T)
    bcat = jnp.zeros((1, 128), jnp.float32).at[0, 16:19].set(b2l)

    z128 = jnp.zeros((CH1, HD), jnp.float32)
    z16 = jnp.zeros((CH1, 16), jnp.float32)
    z16b = jnp.zeros((CH, 16), jnp.float32)
    ones16 = jnp.ones((CH1, 16), jnp.float32)

    y1a, y1b, r1 = _tc1(x, w1lt, w1rt, b1l2d)
    agga, aggb, dega, degb = _sc1(y1a, y1b, src2da, dst2da, z128, z16, ones16)
    y2, base = _tc2(agga, aggb, dega, degb, r1, w2cat, bcat)
    agg2a, agg2b = _sc2(y2, src2db, dst2db, z16b)
    return _tc3(agg2a, agg2b, dega, degb, base)


# speedup vs baseline: 1.0466x; 1.0009x over previous
import jax
import jax.numpy as jnp
from jax import lax
from jax.experimental import pallas as pl
from jax.experimental.pallas import tpu as pltpu
from jax.experimental.pallas import tpu_sc as plsc

N = 10000
NP = 10240
E = 160000
EP = 163840
D = 256
HD = 128
CH = 128
CH1 = 32
G = 32
RPW = NP // 16
NCH1 = EP // 16 // CH1
NG1 = NCH1 // G
NCH2 = EP // 32 // CH
RB = 1000


def _tc1_body(x_ref, w1lt_ref, w1rt_ref, b1l_ref, y1a_ref, y1b_ref, r1_ref):
    xb = x_ref[...]
    y1 = jnp.dot(xb, w1lt_ref[...], preferred_element_type=jnp.float32)
    y1a_ref[...] = y1[:, :HD]
    y1b_ref[...] = y1[:, HD:]
    r1_ref[...] = (
        jnp.dot(xb, w1rt_ref[...], preferred_element_type=jnp.float32)
        + b1l_ref[...]
    )


def _tc1(x, w1lt, w1rt, b1l2d):
    return pl.pallas_call(
        _tc1_body,
        grid=(N // RB,),
        in_specs=[
            pl.BlockSpec((RB, D), lambda i: (i, 0)),
            pl.BlockSpec((D, D), lambda i: (0, 0)),
            pl.BlockSpec((D, D), lambda i: (0, 0)),
            pl.BlockSpec((1, D), lambda i: (0, 0)),
        ],
        out_specs=[
            pl.BlockSpec((RB, HD), lambda i: (i, 0)),
            pl.BlockSpec((RB, HD), lambda i: (i, 0)),
            pl.BlockSpec((RB, D), lambda i: (i, 0)),
        ],
        out_shape=[
            jax.ShapeDtypeStruct((N, HD), jnp.float32),
            jax.ShapeDtypeStruct((N, HD), jnp.float32),
            jax.ShapeDtypeStruct((N, D), jnp.float32),
        ],
    )(x, w1lt, w1rt, b1l2d)


def _sc1_body(y1a, y1b, src2d, dst2d, z128, z16, ones16,
              agga_o, aggb_o, dega_o, degb_o,
              agg_sh, deg_sh, sblk, dblk, buf0, buf1, buf2, buf3, ones_v,
              isem, gsem0, gsem1, gsem2, gsem3, ssem0, ssem1, ssem2, ssem3,
              dsem):
    c = lax.axis_index("c")
    s = lax.axis_index("s")
    rbase = s * RPW
    pltpu.sync_copy(z128, buf0)
    pltpu.sync_copy(z16, ones_v)
    for j in range(RPW // CH1):
        pltpu.sync_copy(buf0, agg_sh.at[pl.ds(rbase + j * CH1, CH1)])
        pltpu.sync_copy(ones_v, deg_sh.at[pl.ds(rbase + j * CH1, CH1)])

    pltpu.sync_copy(ones16, ones_v)
    cb = s * NCH1
    plsc.subcore_barrier()

    bufs = (buf0, buf1, buf2, buf3)
    gsems = (gsem0, gsem1, gsem2, gsem3)
    ssems = (ssem0, ssem1, ssem2, ssem3)

    def run(table):
        def body(gi, carry):
            gb = cb + gi * G
            di1 = pltpu.async_copy(src2d.at[pl.ds(gb, G)], sblk, isem)
            di2 = pltpu.async_copy(dst2d.at[pl.ds(gb, G)], dblk, isem)
            di1.wait()
            di2.wait()

            @pl.when((gi % 2) == c)
            def _():
                for j in range(G):
                    pltpu.async_copy(ones_v, deg_sh.at[dblk.at[j]], dsem,
                                     add=True)

            gds = {}
            sds = {}
            gds[0] = pltpu.async_copy(table.at[sblk.at[0]], bufs[0], gsems[0])
            gds[1] = pltpu.async_copy(table.at[sblk.at[1]], bufs[1], gsems[1])
            for j in range(G):
                if j + 2 < G:
                    if j - 2 >= 0:
                        sds[j - 2].wait()
                    r = (j + 2) % 4
                    gds[j + 2] = pltpu.async_copy(table.at[sblk.at[j + 2]],
                                                  bufs[r], gsems[r])
                gds[j].wait()
                sds[j] = pltpu.async_copy(bufs[j % 4],
                                          agg_sh.at[dblk.at[j]],
                                          ssems[j % 4], add=True)
            sds[G - 4].wait()
            sds[G - 3].wait()
            sds[G - 2].wait()
            sds[G - 1].wait()

            @pl.when((gi % 2) == c)
            def _():
                for j in range(G):
                    pltpu.make_async_copy(z16, ones_v, dsem).wait()

            return carry

        lax.fori_loop(0, NG1, body, 0)

    @pl.when(c == 0)
    def _():
        run(y1a)

    @pl.when(c == 1)
    def _():
        run(y1b)

    plsc.subcore_barrier()
    for j in range(RPW // CH1):
        rs = rbase + j * CH1
        pltpu.sync_copy(agg_sh.at[pl.ds(rs, CH1)], buf0)

        @pl.when(c == 0)
        def _():
            pltpu.sync_copy(deg_sh.at[pl.ds(rs, CH1)], ones_v)
            pltpu.sync_copy(ones_v, dega_o.at[pl.ds(rs, CH1)])
            pltpu.sync_copy(buf0, agga_o.at[pl.ds(rs, CH1)])

        @pl.when(c == 1)
        def _():
            pltpu.sync_copy(deg_sh.at[pl.ds(rs, CH1)], ones_v)
            pltpu.sync_copy(ones_v, degb_o.at[pl.ds(rs, CH1)])
            pltpu.sync_copy(buf0, aggb_o.at[pl.ds(rs, CH1)])


def _sc1(y1a, y1b, src2d, dst2d, z128, z16, ones16):
    mesh = plsc.VectorSubcoreMesh(core_axis_name="c", subcore_axis_name="s", num_cores=2, num_subcores=16)
    return pl.kernel(
        _sc1_body,
        out_type=[
            jax.ShapeDtypeStruct((NP, HD), jnp.float32),
            jax.ShapeDtypeStruct((NP, HD), jnp.float32),
            jax.ShapeDtypeStruct((NP, 16), jnp.float32),
            jax.ShapeDtypeStruct((NP, 16), jnp.float32),
        ],
        mesh=mesh,
        scratch_types=[
            pltpu.VMEM_SHARED((NP, HD), jnp.float32),
            pltpu.VMEM_SHARED((NP, 16), jnp.float32),
            pltpu.VMEM((G, CH1), jnp.int32),
            pltpu.VMEM((G, CH1), jnp.int32),
            pltpu.VMEM((CH1, HD), jnp.float32),
            pltpu.VMEM((CH1, HD), jnp.float32),
            pltpu.VMEM((CH1, HD), jnp.float32),
            pltpu.VMEM((CH1, HD), jnp.float32),
            pltpu.VMEM((CH1, 16), jnp.float32),
            pltpu.SemaphoreType.DMA,
            pltpu.SemaphoreType.DMA,
            pltpu.SemaphoreType.DMA,
            pltpu.SemaphoreType.DMA,
            pltpu.SemaphoreType.DMA,
            pltpu.SemaphoreType.DMA,
            pltpu.SemaphoreType.DMA,
            pltpu.SemaphoreType.DMA,
            pltpu.SemaphoreType.DMA,
            pltpu.SemaphoreType.DMA,
        ],
        compiler_params=pltpu.CompilerParams(use_tc_tiling_on_sc=False),
    )(y1a, y1b, src2d, dst2d, z128, z16, ones16)


def _tc2_body(agga_ref, aggb_ref, dega_ref, degb_ref, r1_ref, w2cat_ref,
              bcat_ref, y2_ref, base_ref):
    rdeg = 1.0 / jnp.maximum(dega_ref[:, 0:1] + degb_ref[:, 0:1], 1.0)
    h0 = jnp.maximum(agga_ref[...] * rdeg + r1_ref[:, :HD], 0.0)
    h1 = jnp.maximum(aggb_ref[...] * rdeg + r1_ref[:, HD:], 0.0)
    res = (
        jnp.dot(h0, w2cat_ref[:HD, :], preferred_element_type=jnp.float32)
        + jnp.dot(h1, w2cat_ref[HD:, :], preferred_element_type=jnp.float32)
        + bcat_ref[...]
    )
    y2_ref[...] = res[:, :16]
    base_ref[...] = res[:, 16:32]


def _tc2(agga, aggb, dega, degb, r1, w2cat, bcat):
    return pl.pallas_call(
        _tc2_body,
        grid=(N // RB,),
        in_specs=[
            pl.BlockSpec((RB, HD), lambda i: (i, 0)),
            pl.BlockSpec((RB, HD), lambda i: (i, 0)),
            pl.BlockSpec((RB, 16), lambda i: (i, 0)),
            pl.BlockSpec((RB, 16), lambda i: (i, 0)),
            pl.BlockSpec((RB, D), lambda i: (i, 0)),
            pl.BlockSpec((D, 128), lambda i: (0, 0)),
            pl.BlockSpec((1, 128), lambda i: (0, 0)),
        ],
        out_specs=[
            pl.BlockSpec((RB, 16), lambda i: (i, 0)),
            pl.BlockSpec((RB, 16), lambda i: (i, 0)),
        ],
        out_shape=[
            jax.ShapeDtypeStruct((N, 16), jnp.float32),
            jax.ShapeDtypeStruct((N, 16), jnp.float32),
        ],
    )(agga, aggb, dega, degb, r1, w2cat, bcat)


def _sc2_body(y2, src2d, dst2d, z16,
              agg2a_o, agg2b_o,
              agg_sh, sblk, dblk, b0, b1, b2, b3,
              isem, qsem0, qsem1, qsem2, qsem3, tsem0, tsem1, tsem2, tsem3):
    c = lax.axis_index("c")
    s = lax.axis_index("s")
    rbase = s * RPW
    pltpu.sync_copy(z16, b0)
    for j in range(RPW // CH):
        pltpu.sync_copy(b0, agg_sh.at[pl.ds(rbase + j * CH, CH)])
    w = c * 16 + s
    cb = w * NCH2
    di1 = pltpu.async_copy(src2d.at[pl.ds(cb, NCH2)], sblk, isem)
    di2 = pltpu.async_copy(dst2d.at[pl.ds(cb, NCH2)], dblk, isem)
    di1.wait()
    di2.wait()
    plsc.subcore_barrier()

    bufs = (b0, b1, b2, b3)
    qsems = (qsem0, qsem1, qsem2, qsem3)
    tsems = (tsem0, tsem1, tsem2, tsem3)
    R = 4
    K = 2
    gds = {}
    sds = {}
    for m in range(K):
        gds[m] = pltpu.async_copy(y2.at[sblk.at[m]], bufs[m % R], qsems[m % R])
    for j in range(NCH2):
        t = j + K
        if t < NCH2:
            if t - R >= 0:
                sds[t - R].wait()
            gds[t] = pltpu.async_copy(y2.at[sblk.at[t]], bufs[t % R],
                                      qsems[t % R])
        gds[j].wait()
        sds[j] = pltpu.async_copy(bufs[j % R], agg_sh.at[dblk.at[j]],
                                  tsems[j % R], add=True)
    for m in range(max(0, NCH2 - R), NCH2):
        sds[m].wait()
    plsc.subcore_barrier()

    for j in range(RPW // CH):
        rs = rbase + j * CH
        pltpu.sync_copy(agg_sh.at[pl.ds(rs, CH)], b0)

        @pl.when(c == 0)
        def _():
            pltpu.sync_copy(b0, agg2a_o.at[pl.ds(rs, CH)])

        @pl.when(c == 1)
        def _():
            pltpu.sync_copy(b0, agg2b_o.at[pl.ds(rs, CH)])


def _sc2(y2, src2d, dst2d, z16):
    mesh = plsc.VectorSubcoreMesh(core_axis_name="c", subcore_axis_name="s", num_cores=2, num_subcores=16)
    return pl.kernel(
        _sc2_body,
        out_type=[
            jax.ShapeDtypeStruct((NP, 16), jnp.float32),
            jax.ShapeDtypeStruct((NP, 16), jnp.float32),
        ],
        mesh=mesh,
        scratch_types=[
            pltpu.VMEM_SHARED((NP, 16), jnp.float32),
            pltpu.VMEM((NCH2, CH), jnp.int32),
            pltpu.VMEM((NCH2, CH), jnp.int32),
            pltpu.VMEM((CH, 16), jnp.float32),
            pltpu.VMEM((CH, 16), jnp.float32),
            pltpu.VMEM((CH, 16), jnp.float32),
            pltpu.VMEM((CH, 16), jnp.float32),
        ] + [pltpu.SemaphoreType.DMA] * 9,
        compiler_params=pltpu.CompilerParams(use_tc_tiling_on_sc=False),
    )(y2, src2d, dst2d, z16)


def _tc3_body(a2a_ref, a2b_ref, dega_ref, degb_ref, base_ref, out_ref):
    rdeg = 1.0 / jnp.maximum(dega_ref[:, 0:1] + degb_ref[:, 0:1], 1.0)
    o = (a2a_ref[...] + a2b_ref[...]) * rdeg + base_ref[...]
    sg = jax.nn.sigmoid(o)
    fsi = jnp.maximum(o[:, 0:1], 0.0) + sg[:, 1:2]
    out_ref[...] = jnp.concatenate([fsi, sg[:, 1:2], sg[:, 2:3]], axis=1)


def _tc3(a2a, a2b, dega, degb, base):
    return pl.pallas_call(
        _tc3_body,
        grid=(N // RB,),
        in_specs=[
            pl.BlockSpec((RB, 16), lambda i: (i, 0)),
            pl.BlockSpec((RB, 16), lambda i: (i, 0)),
            pl.BlockSpec((RB, 16), lambda i: (i, 0)),
            pl.BlockSpec((RB, 16), lambda i: (i, 0)),
            pl.BlockSpec((RB, 16), lambda i: (i, 0)),
        ],
        out_specs=pl.BlockSpec((RB, 3), lambda i: (i, 0)),
        out_shape=jax.ShapeDtypeStruct((N, 3), jnp.float32),
    )(a2a, a2b, dega, degb, base)


def kernel(x, edge_index, W1l, b1l, W1r, W2l, b2l, W2r):
    src = edge_index[0].astype(jnp.int32)
    dst = edge_index[1].astype(jnp.int32)
    pad = EP - E
    srcp = jnp.concatenate([src, jnp.zeros((pad,), jnp.int32)])
    dstp = jnp.concatenate([dst, jnp.full((pad,), N, jnp.int32)])
    src2da = srcp.reshape(EP // CH1, CH1)
    dst2da = dstp.reshape(EP // CH1, CH1)
    src2db = srcp.reshape(EP // CH, CH)
    dst2db = dstp.reshape(EP // CH, CH)

    w1lt = W1l.T
    w1rt = W1r.T
    b1l2d = b1l[None, :]
    w2cat = jnp.zeros((D, 128), jnp.float32)
    w2cat = w2cat.at[:, 0:3].set(W2l.T).at[:, 16:19].set(W2r.T)
    bcat = jnp.zeros((1, 128), jnp.float32).at[0, 16:19].set(b2l)

    z128 = jnp.zeros((CH1, HD), jnp.float32)
    z16 = jnp.zeros((CH1, 16), jnp.float32)
    z16b = jnp.zeros((CH, 16), jnp.float32)
    ones16 = jnp.ones((CH1, 16), jnp.float32)

    y1a, y1b, r1 = _tc1(x, w1lt, w1rt, b1l2d)
    agga, aggb, dega, degb = _sc1(y1a, y1b, src2da, dst2da, z128, z16, ones16)
    y2, base = _tc2(agga, aggb, dega, degb, r1, w2cat, bcat)
    agg2a, agg2b = _sc2(y2, src2db, dst2db, z16b)
    return _tc3(agg2a, agg2b, dega, degb, base)
